# Initial kernel scaffold; baseline (speedup 1.0000x reference)
import functools

import jax
import jax.numpy as jnp
from jax import lax
from jax.experimental import pallas as pl
from jax.experimental.pallas import tpu as pltpu
from jax.experimental.pallas import tpu_sc as plsc

_NC = 2
_NS = 16
_L = 16
_NW = _NC * _NS


@functools.partial(jax.jit, static_argnums=(4, 5))
def _sc_mask(heads, tails, gids, starts, n_nodes, n_edges):
    epw = n_edges // _NW
    mesh = plsc.VectorSubcoreMesh(core_axis_name="c", subcore_axis_name="s")

    @functools.partial(
        pl.kernel,
        mesh=mesh,
        out_type=jax.ShapeDtypeStruct((n_edges,), jnp.int32),
        scratch_types=[
            pltpu.VMEM((n_nodes,), jnp.int32),
            pltpu.VMEM((n_nodes,), jnp.int32),
            pltpu.VMEM((epw,), jnp.int32),
            pltpu.VMEM((epw,), jnp.int32),
            pltpu.VMEM((epw,), jnp.int32),
        ],
    )
    def k(heads_hbm, tails_hbm, gids_hbm, starts_hbm, out_hbm,
          start_tab, ok_tab, h_v, t_v, o_v):
        cid = lax.axis_index("c")
        sid = lax.axis_index("s")
        wid = sid * _NC + cid

        zeros = jnp.zeros((_L,), jnp.int32)
        ones = jnp.full((_L,), 1, jnp.int32)
        neg1 = jnp.full((_L,), -1, jnp.int32)

        pltpu.sync_copy(gids_hbm, ok_tab)

        def init_body(i, carry):
            g = ok_tab[pl.ds(i * _L, _L)]
            ok_tab[pl.ds(i * _L, _L)] = jnp.where(g != neg1, ones, zeros)
            start_tab[pl.ds(i * _L, _L)] = zeros
            return carry

        lax.fori_loop(0, n_nodes // _L, init_body, 0)

        n_chunks = -(-n_nodes // epw)
        for c in range(n_chunks):
            clen = min(epw, n_nodes - c * epw)
            pltpu.sync_copy(starts_hbm.at[pl.ds(c * epw, clen)],
                            o_v.at[pl.ds(0, clen)])

            def scat_body(i, carry):
                idx = o_v[pl.ds(i * _L, _L)]
                plsc.store_scatter(start_tab, [idx], ones)
                return carry

            lax.fori_loop(0, clen // _L, scat_body, 0)

        base = wid * epw
        pltpu.sync_copy(heads_hbm.at[pl.ds(base, epw)], h_v)
        pltpu.sync_copy(tails_hbm.at[pl.ds(base, epw)], t_v)

        def edge_body(i, carry):
            h = h_v[pl.ds(i * _L, _L)]
            t = t_v[pl.ds(i * _L, _L)]
            s = plsc.load_gather(start_tab, [h])
            o = plsc.load_gather(ok_tab, [t])
            o_v[pl.ds(i * _L, _L)] = s & o
            return carry

        lax.fori_loop(0, epw // _L, edge_body, 0)

        pltpu.sync_copy(o_v, out_hbm.at[pl.ds(base, epw)])

    return k(heads, tails, gids, starts)


def kernel(edge_index, edge_batch, node_global_ids, node_ptr, edge_ptr,
           start_node_locals, start_ptr, start_entity_ids, start_entity_ptr,
           answer_node_locals, answer_ptr, answer_entity_ids,
           edge_relations, edge_labels, is_answer_reachable):
    n_edges = edge_index.shape[1]
    n_nodes = node_global_ids.shape[0]
    heads = edge_index[0].astype(jnp.int32)
    tails = edge_index[1].astype(jnp.int32)
    gids = node_global_ids.astype(jnp.int32)
    starts = start_node_locals.astype(jnp.int32)
    out = _sc_mask(heads, tails, gids, starts, n_nodes, n_edges)
    return out != 0


# speedup vs baseline: 650.1619x; 650.1619x over previous
import functools

import jax
import jax.numpy as jnp
from jax import lax
from jax.experimental import pallas as pl
from jax.experimental.pallas import tpu as pltpu
from jax.experimental.pallas import tpu_sc as plsc

_NC = 2
_NS = 16
_L = 16
_NW = _NC * _NS


@functools.partial(jax.jit, static_argnums=(4, 5))
def _sc_mask(heads, tails, gids, starts, n_nodes, n_edges):
    epw = n_edges // _NW
    mesh = plsc.VectorSubcoreMesh(core_axis_name="c", subcore_axis_name="s")

    @functools.partial(
        pl.kernel,
        mesh=mesh,
        compiler_params=pltpu.CompilerParams(needs_layout_passes=False),
        out_type=jax.ShapeDtypeStruct((n_edges,), jnp.int32),
        scratch_types=[
            pltpu.VMEM((n_nodes,), jnp.int32),
            pltpu.VMEM((n_nodes,), jnp.int32),
            pltpu.VMEM((epw,), jnp.int32),
            pltpu.VMEM((epw,), jnp.int32),
            pltpu.VMEM((epw,), jnp.int32),
        ],
    )
    def k(heads_hbm, tails_hbm, gids_hbm, starts_hbm, out_hbm,
          start_tab, ok_tab, h_v, t_v, o_v):
        cid = lax.axis_index("c")
        sid = lax.axis_index("s")
        wid = sid * _NC + cid

        zeros = jnp.zeros((_L,), jnp.int32)
        ones = jnp.full((_L,), 1, jnp.int32)
        neg1 = jnp.full((_L,), -1, jnp.int32)

        pltpu.sync_copy(gids_hbm, ok_tab)

        def init_body(i, carry):
            g = ok_tab[pl.ds(i * _L, _L)]
            ok_tab[pl.ds(i * _L, _L)] = jnp.where(g != neg1, ones, zeros)
            start_tab[pl.ds(i * _L, _L)] = zeros
            return carry

        lax.fori_loop(0, n_nodes // _L, init_body, 0)

        n_chunks = -(-n_nodes // epw)
        for c in range(n_chunks):
            clen = min(epw, n_nodes - c * epw)
            pltpu.sync_copy(starts_hbm.at[pl.ds(c * epw, clen)],
                            o_v.at[pl.ds(0, clen)])

            def scat_body(i, carry):
                idx = o_v[pl.ds(i * _L, _L)]
                plsc.store_scatter(start_tab, [idx], ones)
                return carry

            lax.fori_loop(0, clen // _L, scat_body, 0)

        base = wid * epw
        pltpu.sync_copy(heads_hbm.at[pl.ds(base, epw)], h_v)
        pltpu.sync_copy(tails_hbm.at[pl.ds(base, epw)], t_v)

        def edge_body(i, carry):
            h = h_v[pl.ds(i * _L, _L)]
            t = t_v[pl.ds(i * _L, _L)]
            s = plsc.load_gather(start_tab, [h])
            o = plsc.load_gather(ok_tab, [t])
            o_v[pl.ds(i * _L, _L)] = s & o
            return carry

        lax.fori_loop(0, epw // _L, edge_body, 0)

        pltpu.sync_copy(o_v, out_hbm.at[pl.ds(base, epw)])

    return k(heads, tails, gids, starts)


def kernel(edge_index, edge_batch, node_global_ids, node_ptr, edge_ptr,
           start_node_locals, start_ptr, start_entity_ids, start_entity_ptr,
           answer_node_locals, answer_ptr, answer_entity_ids,
           edge_relations, edge_labels, is_answer_reachable):
    n_edges = edge_index.shape[1]
    n_nodes = node_global_ids.shape[0]
    heads = edge_index[0].astype(jnp.int32)
    tails = edge_index[1].astype(jnp.int32)
    gids = node_global_ids.astype(jnp.int32)
    starts = start_node_locals.astype(jnp.int32)
    out = _sc_mask(heads, tails, gids, starts, n_nodes, n_edges)
    return out != 0


# speedup vs baseline: 710.1981x; 1.0923x over previous
import functools

import jax
import jax.numpy as jnp
from jax import lax
from jax.experimental import pallas as pl
from jax.experimental.pallas import tpu as pltpu
from jax.experimental.pallas import tpu_sc as plsc

_NC = 2
_NS = 16
_L = 16
_NW = _NC * _NS


@functools.partial(jax.jit, static_argnums=(5, 6))
def _sc_mask(heads, tails, gids, starts, zeros_hbm, n_nodes, n_edges):
    epw = n_edges // _NW
    mesh = plsc.VectorSubcoreMesh(core_axis_name="c", subcore_axis_name="s")

    @functools.partial(
        pl.kernel,
        mesh=mesh,
        compiler_params=pltpu.CompilerParams(needs_layout_passes=False),
        out_type=jax.ShapeDtypeStruct((n_edges,), jnp.int32),
        scratch_types=[
            pltpu.VMEM((n_nodes,), jnp.int32),
            pltpu.VMEM((n_nodes,), jnp.int32),
            pltpu.VMEM((epw,), jnp.int32),
            pltpu.VMEM((epw,), jnp.int32),
            pltpu.VMEM((epw,), jnp.int32),
        ],
    )
    def k(heads_hbm, tails_hbm, gids_hbm, starts_hbm, z_hbm, out_hbm,
          start_tab, gid_tab, h_v, t_v, o_v):
        cid = lax.axis_index("c")
        sid = lax.axis_index("s")
        wid = sid * _NC + cid

        zeros = jnp.zeros((_L,), jnp.int32)
        ones = jnp.full((_L,), 1, jnp.int32)
        neg1 = jnp.full((_L,), -1, jnp.int32)

        pltpu.sync_copy(gids_hbm, gid_tab)
        pltpu.sync_copy(z_hbm, start_tab)

        SCAT_U = 8
        n_chunks = -(-n_nodes // epw)
        for c in range(n_chunks):
            clen = min(epw, n_nodes - c * epw)
            pltpu.sync_copy(starts_hbm.at[pl.ds(c * epw, clen)],
                            o_v.at[pl.ds(0, clen)])

            def scat_body(i, carry):
                for u in range(SCAT_U):
                    idx = o_v[pl.ds((i * SCAT_U + u) * _L, _L)]
                    plsc.store_scatter(start_tab, [idx], ones)
                return carry

            lax.fori_loop(0, clen // (_L * SCAT_U), scat_body, 0)

        base = wid * epw
        pltpu.sync_copy(heads_hbm.at[pl.ds(base, epw)], h_v)
        pltpu.sync_copy(tails_hbm.at[pl.ds(base, epw)], t_v)

        EDGE_U = 8

        def edge_body(i, carry):
            for u in range(EDGE_U):
                off = (i * EDGE_U + u) * _L
                h = h_v[pl.ds(off, _L)]
                t = t_v[pl.ds(off, _L)]
                s = plsc.load_gather(start_tab, [h])
                g = plsc.load_gather(gid_tab, [t])
                o_v[pl.ds(off, _L)] = s & jnp.where(g != neg1, ones, zeros)
            return carry

        lax.fori_loop(0, epw // (_L * EDGE_U), edge_body, 0)

        pltpu.sync_copy(o_v, out_hbm.at[pl.ds(base, epw)])

    return k(heads, tails, gids, starts, zeros_hbm)


def kernel(edge_index, edge_batch, node_global_ids, node_ptr, edge_ptr,
           start_node_locals, start_ptr, start_entity_ids, start_entity_ptr,
           answer_node_locals, answer_ptr, answer_entity_ids,
           edge_relations, edge_labels, is_answer_reachable):
    n_edges = edge_index.shape[1]
    n_nodes = node_global_ids.shape[0]
    heads = edge_index[0].astype(jnp.int32)
    tails = edge_index[1].astype(jnp.int32)
    gids = node_global_ids.astype(jnp.int32)
    starts = start_node_locals.astype(jnp.int32)
    zeros_hbm = jnp.zeros((n_nodes,), jnp.int32)
    out = _sc_mask(heads, tails, gids, starts, zeros_hbm, n_nodes, n_edges)
    return out != 0


# speedup vs baseline: 872.8976x; 1.2291x over previous
import functools

import jax
import jax.numpy as jnp
from jax import lax
from jax.experimental import pallas as pl
from jax.experimental.pallas import tpu as pltpu
from jax.experimental.pallas import tpu_sc as plsc

_NC = 2
_NS = 16
_L = 16
_NW = _NC * _NS
_ROW = 128


@functools.partial(jax.jit, static_argnums=(5, 6))
def _sc_mask(heads, tails, gids, starts, zeros_hbm, n_nodes, n_edges):
    epw = n_edges // _NW
    npc = n_nodes // _NS
    nrows = npc // _ROW
    mesh = plsc.VectorSubcoreMesh(core_axis_name="c", subcore_axis_name="s")

    @functools.partial(
        pl.kernel,
        mesh=mesh,
        compiler_params=pltpu.CompilerParams(needs_layout_passes=False),
        out_type=jax.ShapeDtypeStruct((n_edges,), jnp.int32),
        scratch_types=[
            pltpu.VMEM((n_nodes,), jnp.int32),
            pltpu.VMEM((n_nodes,), jnp.int32),
            pltpu.VMEM((epw,), jnp.int32),
            pltpu.VMEM((epw,), jnp.int32),
            pltpu.VMEM((epw,), jnp.int32),
            pltpu.VMEM((nrows, _ROW), jnp.int32),
            pltpu.VMEM((_ROW,), jnp.int32),
            pltpu.VMEM_SHARED((n_nodes,), jnp.int32),
            pltpu.VMEM_SHARED((n_nodes,), jnp.int32),
            pltpu.SemaphoreType.DMA,
        ],
    )
    def k(heads_hbm, tails_hbm, gids_hbm, starts_hbm, z_hbm, out_hbm,
          start_tab, gid_tab, h_v, t_v, o_v, idx2, ones_v,
          start_sp, gid_sp, sem):
        cid = lax.axis_index("c")
        sid = lax.axis_index("s")
        wid = sid * _NC + cid
        base = wid * epw

        cp_h = pltpu.make_async_copy(heads_hbm.at[pl.ds(base, epw)], h_v, sem)
        cp_h.start()
        cp_t = pltpu.make_async_copy(tails_hbm.at[pl.ds(base, epw)], t_v, sem)
        cp_t.start()

        soff = sid * npc
        pltpu.sync_copy(z_hbm.at[pl.ds(soff, npc)],
                        start_sp.at[pl.ds(soff, npc)])
        pltpu.sync_copy(gids_hbm.at[pl.ds(soff, npc)],
                        gid_sp.at[pl.ds(soff, npc)])
        for j in range(nrows):
            pltpu.sync_copy(starts_hbm.at[pl.ds(soff + j * _ROW, _ROW)],
                            idx2.at[j])
        ones = jnp.full((_L,), 1, jnp.int32)
        for j in range(_ROW // _L):
            ones_v[pl.ds(j * _L, _L)] = ones

        plsc.subcore_barrier()
        for j in range(nrows):
            pltpu.sync_copy(ones_v, start_sp.at[idx2.at[j]], add=True)
        plsc.subcore_barrier()

        pltpu.sync_copy(start_sp, start_tab)
        pltpu.sync_copy(gid_sp, gid_tab)

        cp_h.wait()
        cp_t.wait()

        zeros = jnp.zeros((_L,), jnp.int32)
        onesl = jnp.full((_L,), 1, jnp.int32)
        neg1 = jnp.full((_L,), -1, jnp.int32)
        EDGE_U = 8

        def edge_body(i, carry):
            for u in range(EDGE_U):
                off = (i * EDGE_U + u) * _L
                h = h_v[pl.ds(off, _L)]
                t = t_v[pl.ds(off, _L)]
                s = plsc.load_gather(start_tab, [h])
                g = plsc.load_gather(gid_tab, [t])
                m = (s != zeros) & (g != neg1)
                o_v[pl.ds(off, _L)] = jnp.where(m, onesl, zeros)
            return carry

        lax.fori_loop(0, epw // (_L * EDGE_U), edge_body, 0)

        pltpu.sync_copy(o_v, out_hbm.at[pl.ds(base, epw)])

    return k(heads, tails, gids, starts, zeros_hbm)


def kernel(edge_index, edge_batch, node_global_ids, node_ptr, edge_ptr,
           start_node_locals, start_ptr, start_entity_ids, start_entity_ptr,
           answer_node_locals, answer_ptr, answer_entity_ids,
           edge_relations, edge_labels, is_answer_reachable):
    n_edges = edge_index.shape[1]
    n_nodes = node_global_ids.shape[0]
    heads = edge_index[0].astype(jnp.int32)
    tails = edge_index[1].astype(jnp.int32)
    gids = node_global_ids.astype(jnp.int32)
    starts = start_node_locals.astype(jnp.int32)
    zeros_hbm = jnp.zeros((n_nodes,), jnp.int32)
    out = _sc_mask(heads, tails, gids, starts, zeros_hbm, n_nodes, n_edges)
    return out != 0


# speedup vs baseline: 1072.0642x; 1.2282x over previous
import functools

import jax
import jax.numpy as jnp
from jax import lax
from jax.experimental import pallas as pl
from jax.experimental.pallas import tpu as pltpu
from jax.experimental.pallas import tpu_sc as plsc

_NC = 2
_NS = 16
_L = 16
_NW = _NC * _NS
_ROW = 128


@functools.partial(jax.jit, static_argnums=(5, 6))
def _sc_mask(heads, tails, gids, starts2d, zeros_hbm, n_nodes, n_edges):
    epw = n_edges // _NW
    npc = n_nodes // _NS
    nrows = npc // _ROW
    mesh = plsc.VectorSubcoreMesh(core_axis_name="c", subcore_axis_name="s")

    @functools.partial(
        pl.kernel,
        mesh=mesh,
        compiler_params=pltpu.CompilerParams(needs_layout_passes=False),
        out_type=jax.ShapeDtypeStruct((n_edges,), jnp.int32),
        scratch_types=[
            pltpu.VMEM((n_nodes,), jnp.int32),
            pltpu.VMEM((n_nodes,), jnp.int32),
            pltpu.VMEM((epw,), jnp.int32),
            pltpu.VMEM((epw,), jnp.int32),
            pltpu.VMEM((epw,), jnp.int32),
            pltpu.VMEM((nrows, _ROW), jnp.int32),
            pltpu.VMEM((_ROW,), jnp.int32),
            pltpu.VMEM_SHARED((n_nodes,), jnp.int32),
            pltpu.VMEM_SHARED((n_nodes,), jnp.int32),
            pltpu.SemaphoreType.DMA,
            pltpu.SemaphoreType.DMA,
        ],
    )
    def k(heads_hbm, tails_hbm, gids_hbm, starts_hbm, z_hbm, out_hbm,
          start_tab, gid_tab, h_v, t_v, o_v, idx2, ones_v,
          start_sp, gid_sp, sem, sem2):
        cid = lax.axis_index("c")
        sid = lax.axis_index("s")
        wid = sid * _NC + cid
        base = wid * epw

        cp_h = pltpu.make_async_copy(heads_hbm.at[pl.ds(base, epw)], h_v, sem)
        cp_h.start()
        cp_t = pltpu.make_async_copy(tails_hbm.at[pl.ds(base, epw)], t_v, sem)
        cp_t.start()

        soff = sid * npc
        st0 = pltpu.async_copy(z_hbm.at[pl.ds(soff, npc)],
                               start_sp.at[pl.ds(soff, npc)], sem2)
        st1 = pltpu.async_copy(gids_hbm.at[pl.ds(soff, npc)],
                               gid_sp.at[pl.ds(soff, npc)], sem2)
        st2 = pltpu.async_copy(starts_hbm.at[pl.ds(sid * nrows, nrows), :],
                               idx2, sem2)
        ones = jnp.full((_L,), 1, jnp.int32)
        for j in range(_ROW // _L):
            ones_v[pl.ds(j * _L, _L)] = ones
        st0.wait()
        st1.wait()
        st2.wait()

        plsc.subcore_barrier()
        scats = [
            pltpu.async_copy(ones_v, start_sp.at[idx2.at[j]], sem2, add=True)
            for j in range(nrows)
        ]
        for s in scats:
            s.wait()
        plsc.subcore_barrier()

        pltpu.sync_copy(start_sp, start_tab)
        pltpu.sync_copy(gid_sp, gid_tab)

        cp_h.wait()
        cp_t.wait()

        zeros = jnp.zeros((_L,), jnp.int32)
        onesl = jnp.full((_L,), 1, jnp.int32)
        neg1 = jnp.full((_L,), -1, jnp.int32)
        EDGE_U = 8

        def edge_body(i, carry):
            for u in range(EDGE_U):
                off = (i * EDGE_U + u) * _L
                h = h_v[pl.ds(off, _L)]
                t = t_v[pl.ds(off, _L)]
                s = plsc.load_gather(start_tab, [h])
                g = plsc.load_gather(gid_tab, [t])
                m = (s != zeros) & (g != neg1)
                o_v[pl.ds(off, _L)] = jnp.where(m, onesl, zeros)
            return carry

        lax.fori_loop(0, epw // (_L * EDGE_U), edge_body, 0)

        pltpu.sync_copy(o_v, out_hbm.at[pl.ds(base, epw)])

    return k(heads, tails, gids, starts2d, zeros_hbm)


def kernel(edge_index, edge_batch, node_global_ids, node_ptr, edge_ptr,
           start_node_locals, start_ptr, start_entity_ids, start_entity_ptr,
           answer_node_locals, answer_ptr, answer_entity_ids,
           edge_relations, edge_labels, is_answer_reachable):
    n_edges = edge_index.shape[1]
    n_nodes = node_global_ids.shape[0]
    heads = edge_index[0].astype(jnp.int32)
    tails = edge_index[1].astype(jnp.int32)
    gids = node_global_ids.astype(jnp.int32)
    starts2d = start_node_locals.astype(jnp.int32).reshape(-1, _ROW)
    zeros_hbm = jnp.zeros((n_nodes,), jnp.int32)
    out = _sc_mask(heads, tails, gids, starts2d, zeros_hbm, n_nodes, n_edges)
    return out != 0
